# baseline (speedup 1.0000x reference)
import functools

import jax
import jax.numpy as jnp
from jax import lax
from jax.experimental import pallas as pl
from jax.experimental.pallas import tpu as pltpu
from jax.experimental.pallas import tpu_sc as plsc

N = 50000
E = 800000
D = 64
H = 32
G = 64
L = 9
EPS = 1e-05
NEG = -1e30

IDXW = 128
CHUNK = 128
TILES = 16
E_PAD = 802816
SUPER = 14
CHUNKS = E_PAD // (TILES * CHUNK)
SUPERS = CHUNKS // SUPER
ROWS_PER_TILE = E_PAD // (TILES * IDXW)
N_PAD = 50176
DUMP = N
NPT = N_PAD // TILES
ZROWS = 56

RBLK = 2000
NBLKS = N // RBLK


def _sc_message(h0, h1, w0, w1, eT, src2d, dst2d):
    mesh = plsc.VectorSubcoreMesh(core_axis_name="c", subcore_axis_name="s")

    @functools.partial(
        pl.kernel,
        mesh=mesh,
        compiler_params=pltpu.CompilerParams(use_tc_tiling_on_sc=False),
        out_type=[jax.ShapeDtypeStruct((N_PAD, H), jnp.float32),
                  jax.ShapeDtypeStruct((N_PAD, H), jnp.float32)],
        scratch_types=[
            pltpu.VMEM((SUPER, IDXW), jnp.int32),
            pltpu.VMEM((SUPER, IDXW), jnp.int32),
            pltpu.VMEM((CHUNK, H), jnp.float32),
            pltpu.VMEM((CHUNK, H), jnp.float32),
            pltpu.VMEM((4, IDXW), jnp.float32),
            pltpu.VMEM((4, IDXW), jnp.float32),
            pltpu.VMEM((4, H), jnp.float32),
            pltpu.VMEM((ZROWS, H), jnp.float32),
            pltpu.VMEM_SHARED((N_PAD, H), jnp.float32),
            pltpu.SemaphoreType.DMA,
            pltpu.SemaphoreType.DMA,
        ],
    )
    def k(h0_hbm, h1_hbm, w0_hbm, w1_hbm, eT_hbm, src_hbm, dst_hbm,
          out0, out1, sidx, didx, g0b, g1b, a0b, a1b, wbuf, zbuf, accum,
          sem0, sem1):
        c = lax.axis_index("c")
        s = lax.axis_index("s")
        gb = (g0b, g1b)
        ab = (a0b, a1b)
        sems = (sem0, sem1)

        def zb(r, carry):
            z = jnp.zeros((16,), jnp.float32)
            zbuf[r, pl.ds(0, 16)] = z
            zbuf[r, pl.ds(16, 16)] = z
            return carry
        lax.fori_loop(0, ZROWS, zb, 0)

        def za(q, carry):
            pltpu.sync_copy(zbuf, accum.at[pl.ds(s * NPT + q * ZROWS, ZROWS)])
            return carry
        lax.fori_loop(0, NPT // ZROWS, za, 0)
        plsc.subcore_barrier()

        def run(h_hbm, w_hbm, out_hbm):
            pltpu.sync_copy(w_hbm, wbuf)
            wv = [[wbuf[kk, pl.ds(hh * 16, 16)] for hh in range(2)]
                  for kk in range(4)]

            def super_body(sj, carry):
                base = s * ROWS_PER_TILE + sj * SUPER
                pltpu.sync_copy(src_hbm.at[pl.ds(base, SUPER)], sidx)
                pltpu.sync_copy(dst_hbm.at[pl.ds(base, SUPER)], didx)

                def issue(cc, b):
                    eo = (base + cc) * IDXW
                    hs = [pltpu.async_copy(h_hbm.at[sidx.at[cc]], gb[b],
                                           sems[b])]
                    for kk in range(4):
                        hs.append(pltpu.async_copy(
                            eT_hbm.at[kk, pl.ds(eo, IDXW)], ab[b].at[kk],
                            sems[b]))
                    return hs

                hnd = {0: issue(0, 0)}
                for cc in range(SUPER):
                    b = cc & 1
                    if cc + 1 < SUPER:
                        hnd[cc + 1] = issue(cc + 1, 1 - b)
                    for hh in hnd.pop(cc):
                        hh.wait()
                    gbuf, abuf = gb[b], ab[b]

                    def rb(it, carry2, gbuf=gbuf, abuf=abuf):
                        rbase = it * 16
                        grp = pl.ds(rbase, 16)
                        av = [abuf[kk, grp] for kk in range(4)]
                        for u in range(16):
                            r = rbase + u
                            bk = [av[kk][u] for kk in range(4)]
                            for half in range(2):
                                e = ((bk[0] * wv[0][half]
                                      + bk[1] * wv[1][half])
                                     + (bk[2] * wv[2][half]
                                        + bk[3] * wv[3][half]))
                                sl = pl.ds(half * 16, 16)
                                gbuf[r, sl] = jnp.maximum(gbuf[r, sl] + e, 0.0)
                        return carry2
                    lax.fori_loop(0, CHUNK // 16, rb, 0)
                    pltpu.sync_copy(gbuf, accum.at[didx.at[cc]], add=True)
                return carry
            lax.fori_loop(0, SUPERS, super_body, 0)
            plsc.subcore_barrier()

            def rb2(q, carry):
                rbase = s * NPT + q * ZROWS
                pltpu.sync_copy(accum.at[pl.ds(rbase, ZROWS)], zbuf)
                pltpu.sync_copy(zbuf, out_hbm.at[pl.ds(rbase, ZROWS)])
                return carry
            lax.fori_loop(0, NPT // ZROWS, rb2, 0)

        @pl.when(c == 0)
        def _():
            run(h0_hbm, w0_hbm, out0)

        @pl.when(c == 1)
        def _():
            run(h1_hbm, w1_hbm, out1)

    return k(h0, h1, w0, w1, eT, src2d, dst2d)


def _embed_body(x_ref, wv_ref, h0_ref, h1_ref):
    h = jnp.dot(x_ref[...], wv_ref[...], preferred_element_type=jnp.float32)
    h0_ref[...] = h[:, :H]
    h1_ref[...] = h[:, H:]


def _tc_embed(x, W_vert):
    return pl.pallas_call(
        _embed_body,
        grid=(NBLKS,),
        in_specs=[pl.BlockSpec((RBLK, 13), lambda i: (i, 0)),
                  pl.BlockSpec((13, D), lambda i: (0, 0))],
        out_specs=[pl.BlockSpec((RBLK, H), lambda i: (i, 0)),
                   pl.BlockSpec((RBLK, H), lambda i: (i, 0))],
        out_shape=[jax.ShapeDtypeStruct((N, H), jnp.float32),
                   jax.ShapeDtypeStruct((N, H), jnp.float32)],
    )(x, W_vert)


def _layer_body(h0_ref, h1_ref, a0_ref, a1_ref, w_ref, b_ref, o0_ref, o1_ref):
    hb = jnp.concatenate([h0_ref[...], h1_ref[...]], axis=1)
    a = jnp.concatenate([a0_ref[...], a1_ref[...]], axis=1)
    y = jnp.dot((1.0 + EPS) * hb + a, w_ref[...],
                preferred_element_type=jnp.float32) + b_ref[...]
    y = jnp.maximum(y, 0.0) + hb
    o0_ref[...] = y[:, :H]
    o1_ref[...] = y[:, H:]


def _tc_layer(h0, h1, a0, a1, w, b):
    return pl.pallas_call(
        _layer_body,
        grid=(NBLKS,),
        in_specs=[pl.BlockSpec((RBLK, H), lambda i: (i, 0)),
                  pl.BlockSpec((RBLK, H), lambda i: (i, 0)),
                  pl.BlockSpec((RBLK, H), lambda i: (i, 0)),
                  pl.BlockSpec((RBLK, H), lambda i: (i, 0)),
                  pl.BlockSpec((D, D), lambda i: (0, 0)),
                  pl.BlockSpec((1, D), lambda i: (0, 0))],
        out_specs=[pl.BlockSpec((RBLK, H), lambda i: (i, 0)),
                   pl.BlockSpec((RBLK, H), lambda i: (i, 0))],
        out_shape=[jax.ShapeDtypeStruct((N, H), jnp.float32),
                   jax.ShapeDtypeStruct((N, H), jnp.float32)],
    )(h0, h1, a0, a1, w, b)


def _head_body(b3_ref, h0_ref, h1_ref, wh1_ref, bh1_ref, wh2_ref, bh2_ref,
               out_ref, sums, cnt):
    i = pl.program_id(0)

    @pl.when(i == 0)
    def _():
        sums[...] = jnp.zeros_like(sums)
        cnt[...] = jnp.zeros_like(cnt)

    hb = jnp.concatenate([h0_ref[...], h1_ref[...]], axis=1)
    bvec = b3_ref[...].reshape(1, RBLK)
    onehot = (lax.broadcasted_iota(jnp.int32, (G, RBLK), 0) == bvec
              ).astype(jnp.float32)
    sums[...] += jnp.dot(onehot, hb, preferred_element_type=jnp.float32)
    cnt[...] += jnp.sum(onehot, axis=1, keepdims=True)

    @pl.when(i == NBLKS - 1)
    def _():
        pooled = sums[...] / jnp.maximum(cnt[...], 1.0)
        z1 = jnp.dot(pooled, wh1_ref[...],
                     preferred_element_type=jnp.float32) + bh1_ref[...]
        z1 = 0.5 * z1 * (1.0 + lax.erf(z1 * (2.0 ** -0.5)))
        out_ref[...] = jnp.dot(z1, wh2_ref[...],
                               preferred_element_type=jnp.float32) + bh2_ref[...]


def _tc_head(batch3, h0, h1, W_h1, b_h1, W_h2, b_h2):
    return pl.pallas_call(
        _head_body,
        grid=(NBLKS,),
        in_specs=[pl.BlockSpec((1, 1, RBLK), lambda i: (i, 0, 0)),
                  pl.BlockSpec((RBLK, H), lambda i: (i, 0)),
                  pl.BlockSpec((RBLK, H), lambda i: (i, 0)),
                  pl.BlockSpec((D, 512), lambda i: (0, 0)),
                  pl.BlockSpec((1, 512), lambda i: (0, 0)),
                  pl.BlockSpec((512, 1), lambda i: (0, 0)),
                  pl.BlockSpec((1, 1), lambda i: (0, 0))],
        out_specs=pl.BlockSpec((G, 1), lambda i: (0, 0)),
        out_shape=jax.ShapeDtypeStruct((G, 1), jnp.float32),
        scratch_shapes=[pltpu.VMEM((G, D), jnp.float32),
                        pltpu.VMEM((G, 1), jnp.float32)],
    )(batch3, h0, h1, W_h1, b_h1, W_h2, b_h2)


def kernel(x, edge_index, edge_attr, batch, W_vert, W_edge, W_conv, b_conv,
           W_h1, b_h1, W_h2, b_h2):
    src = edge_index[0]
    dst = edge_index[1]
    pad = E_PAD - E
    src2d = jnp.pad(src, (0, pad)).reshape(E_PAD // IDXW, IDXW)
    dst2d = jnp.pad(dst, (0, pad), constant_values=DUMP
                    ).reshape(E_PAD // IDXW, IDXW)
    eT = jnp.pad(edge_attr.T, ((0, 0), (0, pad)))
    w0 = W_edge[:, :H]
    w1 = W_edge[:, H:]
    h0, h1 = _tc_embed(x, W_vert)

    for i in range(L):
        a0, a1 = _sc_message(h0, h1, w0, w1, eT, src2d, dst2d)
        h0, h1 = _tc_layer(h0, h1, a0, a1, W_conv[i],
                           b_conv[i].reshape(1, D))

    batch3 = batch.reshape(NBLKS, 1, RBLK)
    return _tc_head(batch3, h0, h1, W_h1, b_h1.reshape(1, 512),
                    W_h2, b_h2.reshape(1, 1))


# speedup vs baseline: 1.1582x; 1.1582x over previous
import functools

import jax
import jax.numpy as jnp
from jax import lax
from jax.experimental import pallas as pl
from jax.experimental.pallas import tpu as pltpu
from jax.experimental.pallas import tpu_sc as plsc

N = 50000
E = 800000
D = 64
H = 32
G = 64
L = 9
EPS = 1e-05
NEG = -1e30

IDXW = 128
CHUNK = 128
TILES = 16
E_PAD = 802816
SUPER = 14
CHUNKS = E_PAD // (TILES * CHUNK)
SUPERS = CHUNKS // SUPER
ROWS_PER_TILE = E_PAD // (TILES * IDXW)
N_PAD = 50176
DUMP = N
NPT = N_PAD // TILES
ZROWS = 56

RBLK = 2000
NBLKS = N // RBLK


def _sc_message(h0, h1, ea0, ea1, src2d, dst2d):
    mesh = plsc.VectorSubcoreMesh(core_axis_name="c", subcore_axis_name="s")

    @functools.partial(
        pl.kernel,
        mesh=mesh,
        compiler_params=pltpu.CompilerParams(use_tc_tiling_on_sc=False),
        out_type=[jax.ShapeDtypeStruct((N_PAD, H), jnp.float32),
                  jax.ShapeDtypeStruct((N_PAD, H), jnp.float32)],
        scratch_types=[
            pltpu.VMEM((SUPER, IDXW), jnp.int32),
            pltpu.VMEM((SUPER, IDXW), jnp.int32),
            pltpu.VMEM((CHUNK, H), jnp.float32),
            pltpu.VMEM((CHUNK, H), jnp.float32),
            pltpu.VMEM((CHUNK // 4, 128), jnp.float32),
            pltpu.VMEM((CHUNK // 4, 128), jnp.float32),
            pltpu.VMEM((ZROWS, H), jnp.float32),
            pltpu.VMEM_SHARED((N_PAD, H), jnp.float32),
            pltpu.SemaphoreType.DMA,
            pltpu.SemaphoreType.DMA,
        ],
    )
    def k(h0_hbm, h1_hbm, ea0_hbm, ea1_hbm, src_hbm, dst_hbm,
          out0, out1, sidx, didx, g0b, g1b, e0b, e1b, zbuf, accum,
          sem0, sem1):
        c = lax.axis_index("c")
        s = lax.axis_index("s")
        gb = (g0b, g1b)
        eab = (e0b, e1b)
        sems = (sem0, sem1)

        def zb(r, carry):
            z = jnp.zeros((16,), jnp.float32)
            zbuf[r, pl.ds(0, 16)] = z
            zbuf[r, pl.ds(16, 16)] = z
            return carry
        lax.fori_loop(0, ZROWS, zb, 0)

        def za(q, carry):
            pltpu.sync_copy(zbuf, accum.at[pl.ds(s * NPT + q * ZROWS, ZROWS)])
            return carry
        lax.fori_loop(0, NPT // ZROWS, za, 0)
        plsc.subcore_barrier()

        def run(h_hbm, ea_hbm, out_hbm):
            def super_body(sj, carry):
                base = s * ROWS_PER_TILE + sj * SUPER
                pltpu.sync_copy(src_hbm.at[pl.ds(base, SUPER)], sidx)
                pltpu.sync_copy(dst_hbm.at[pl.ds(base, SUPER)], didx)

                def issue(cc, b):
                    erow = (base + cc) * (CHUNK // 4)
                    hg = pltpu.async_copy(h_hbm.at[sidx.at[cc]], gb[b],
                                          sems[b])
                    he = pltpu.async_copy(ea_hbm.at[pl.ds(erow, CHUNK // 4)],
                                          eab[b], sems[b])
                    return (hg, he)

                hnd = {0: issue(0, 0)}
                for cc in range(SUPER):
                    b = cc & 1
                    if cc + 1 < SUPER:
                        hnd[cc + 1] = issue(cc + 1, 1 - b)
                    for hh in hnd.pop(cc):
                        hh.wait()
                    gbuf, ebuf = gb[b], eab[b]

                    def rb(it, carry2, gbuf=gbuf, ebuf=ebuf):
                        rbase = it * 8
                        erbase = it * 2
                        for u in range(8):
                            r = rbase + u
                            er = erbase + u // 4
                            for half in range(2):
                                sl = pl.ds(half * 16, 16)
                                esl = pl.ds((u % 4) * 32 + half * 16, 16)
                                gbuf[r, sl] = jnp.maximum(
                                    gbuf[r, sl] + ebuf[er, esl], 0.0)
                        return carry2
                    lax.fori_loop(0, CHUNK // 8, rb, 0)
                    pltpu.sync_copy(gbuf, accum.at[didx.at[cc]], add=True)
                return carry
            lax.fori_loop(0, SUPERS, super_body, 0)
            plsc.subcore_barrier()

            def rb2(q, carry):
                rbase = s * NPT + q * ZROWS
                pltpu.sync_copy(accum.at[pl.ds(rbase, ZROWS)], zbuf)
                pltpu.sync_copy(zbuf, out_hbm.at[pl.ds(rbase, ZROWS)])
                return carry
            lax.fori_loop(0, NPT // ZROWS, rb2, 0)

        @pl.when(c == 0)
        def _():
            run(h0_hbm, ea0_hbm, out0)

        @pl.when(c == 1)
        def _():
            run(h1_hbm, ea1_hbm, out1)

    return k(h0, h1, ea0, ea1, src2d, dst2d)


def _embed_body(x_ref, wv_ref, h0_ref, h1_ref):
    h = jnp.dot(x_ref[...], wv_ref[...], preferred_element_type=jnp.float32)
    h0_ref[...] = h[:, :H]
    h1_ref[...] = h[:, H:]


def _tc_embed(x, W_vert):
    return pl.pallas_call(
        _embed_body,
        grid=(NBLKS,),
        in_specs=[pl.BlockSpec((RBLK, 13), lambda i: (i, 0)),
                  pl.BlockSpec((13, D), lambda i: (0, 0))],
        out_specs=[pl.BlockSpec((RBLK, H), lambda i: (i, 0)),
                   pl.BlockSpec((RBLK, H), lambda i: (i, 0))],
        out_shape=[jax.ShapeDtypeStruct((N, H), jnp.float32),
                   jax.ShapeDtypeStruct((N, H), jnp.float32)],
    )(x, W_vert)


ETBLK = 512


def _ea_body(et_ref, p_ref, wbd0_ref, wbd1_ref, ea0_ref, ea1_ref):
    r_all = lax.dot_general(p_ref[...], et_ref[...],
                            (((1,), (1,)), ((), ())),
                            preferred_element_type=jnp.float32)
    q = ETBLK // 4
    att = jnp.concatenate([r_all[0:q], r_all[q:2 * q],
                           r_all[2 * q:3 * q], r_all[3 * q:]], axis=1)
    ea0_ref[...] = jnp.dot(att, wbd0_ref[...],
                           preferred_element_type=jnp.float32)
    ea1_ref[...] = jnp.dot(att, wbd1_ref[...],
                           preferred_element_type=jnp.float32)


def _tc_edge_embed(eT, P, wbd0, wbd1):
    return pl.pallas_call(
        _ea_body,
        grid=(E_PAD // ETBLK,),
        in_specs=[pl.BlockSpec((4, ETBLK), lambda i: (0, i)),
                  pl.BlockSpec((ETBLK, ETBLK), lambda i: (0, 0)),
                  pl.BlockSpec((16, 128), lambda i: (0, 0)),
                  pl.BlockSpec((16, 128), lambda i: (0, 0))],
        out_specs=[pl.BlockSpec((ETBLK // 4, 128), lambda i: (i, 0)),
                   pl.BlockSpec((ETBLK // 4, 128), lambda i: (i, 0))],
        out_shape=[jax.ShapeDtypeStruct((E_PAD // 4, 128), jnp.float32),
                   jax.ShapeDtypeStruct((E_PAD // 4, 128), jnp.float32)],
    )(eT, P, wbd0, wbd1)


def _layer_body(h0_ref, h1_ref, a0_ref, a1_ref, w_ref, b_ref, o0_ref, o1_ref):
    hb = jnp.concatenate([h0_ref[...], h1_ref[...]], axis=1)
    a = jnp.concatenate([a0_ref[...], a1_ref[...]], axis=1)
    y = jnp.dot((1.0 + EPS) * hb + a, w_ref[...],
                preferred_element_type=jnp.float32) + b_ref[...]
    y = jnp.maximum(y, 0.0) + hb
    o0_ref[...] = y[:, :H]
    o1_ref[...] = y[:, H:]


def _tc_layer(h0, h1, a0, a1, w, b):
    return pl.pallas_call(
        _layer_body,
        grid=(NBLKS,),
        in_specs=[pl.BlockSpec((RBLK, H), lambda i: (i, 0)),
                  pl.BlockSpec((RBLK, H), lambda i: (i, 0)),
                  pl.BlockSpec((RBLK, H), lambda i: (i, 0)),
                  pl.BlockSpec((RBLK, H), lambda i: (i, 0)),
                  pl.BlockSpec((D, D), lambda i: (0, 0)),
                  pl.BlockSpec((1, D), lambda i: (0, 0))],
        out_specs=[pl.BlockSpec((RBLK, H), lambda i: (i, 0)),
                   pl.BlockSpec((RBLK, H), lambda i: (i, 0))],
        out_shape=[jax.ShapeDtypeStruct((N, H), jnp.float32),
                   jax.ShapeDtypeStruct((N, H), jnp.float32)],
    )(h0, h1, a0, a1, w, b)


def _head_body(b3_ref, h0_ref, h1_ref, wh1_ref, bh1_ref, wh2_ref, bh2_ref,
               out_ref, sums, cnt):
    i = pl.program_id(0)

    @pl.when(i == 0)
    def _():
        sums[...] = jnp.zeros_like(sums)
        cnt[...] = jnp.zeros_like(cnt)

    hb = jnp.concatenate([h0_ref[...], h1_ref[...]], axis=1)
    bvec = b3_ref[...].reshape(1, RBLK)
    onehot = (lax.broadcasted_iota(jnp.int32, (G, RBLK), 0) == bvec
              ).astype(jnp.float32)
    sums[...] += jnp.dot(onehot, hb, preferred_element_type=jnp.float32)
    cnt[...] += jnp.sum(onehot, axis=1, keepdims=True)

    @pl.when(i == NBLKS - 1)
    def _():
        pooled = sums[...] / jnp.maximum(cnt[...], 1.0)
        z1 = jnp.dot(pooled, wh1_ref[...],
                     preferred_element_type=jnp.float32) + bh1_ref[...]
        z1 = 0.5 * z1 * (1.0 + lax.erf(z1 * (2.0 ** -0.5)))
        out_ref[...] = jnp.dot(z1, wh2_ref[...],
                               preferred_element_type=jnp.float32) + bh2_ref[...]


def _tc_head(batch3, h0, h1, W_h1, b_h1, W_h2, b_h2):
    return pl.pallas_call(
        _head_body,
        grid=(NBLKS,),
        in_specs=[pl.BlockSpec((1, 1, RBLK), lambda i: (i, 0, 0)),
                  pl.BlockSpec((RBLK, H), lambda i: (i, 0)),
                  pl.BlockSpec((RBLK, H), lambda i: (i, 0)),
                  pl.BlockSpec((D, 512), lambda i: (0, 0)),
                  pl.BlockSpec((1, 512), lambda i: (0, 0)),
                  pl.BlockSpec((512, 1), lambda i: (0, 0)),
                  pl.BlockSpec((1, 1), lambda i: (0, 0))],
        out_specs=pl.BlockSpec((G, 1), lambda i: (0, 0)),
        out_shape=jax.ShapeDtypeStruct((G, 1), jnp.float32),
        scratch_shapes=[pltpu.VMEM((G, D), jnp.float32),
                        pltpu.VMEM((G, 1), jnp.float32)],
    )(batch3, h0, h1, W_h1, b_h1, W_h2, b_h2)


def kernel(x, edge_index, edge_attr, batch, W_vert, W_edge, W_conv, b_conv,
           W_h1, b_h1, W_h2, b_h2):
    src = edge_index[0]
    dst = edge_index[1]
    pad = E_PAD - E
    src2d = jnp.pad(src, (0, pad)).reshape(E_PAD // IDXW, IDXW)
    dst2d = jnp.pad(dst, (0, pad), constant_values=DUMP
                    ).reshape(E_PAD // IDXW, IDXW)
    eT = jnp.pad(edge_attr.T, ((0, 0), (0, pad)))
    q = ETBLK // 4
    rr = jnp.arange(ETBLK)
    targ = 4 * (rr % q) + rr // q
    P = (jnp.arange(ETBLK)[None, :] == targ[:, None]).astype(jnp.float32)
    blk = (jnp.arange(16)[:, None] // 4 == jnp.arange(128)[None, :] // H)
    wbd0 = jnp.where(blk, jnp.tile(W_edge[:, :H], (4, 4)), 0.0)
    wbd1 = jnp.where(blk, jnp.tile(W_edge[:, H:], (4, 4)), 0.0)

    ea0, ea1 = _tc_edge_embed(eT, P, wbd0, wbd1)
    h0, h1 = _tc_embed(x, W_vert)

    for i in range(L):
        a0, a1 = _sc_message(h0, h1, ea0, ea1, src2d, dst2d)
        h0, h1 = _tc_layer(h0, h1, a0, a1, W_conv[i],
                           b_conv[i].reshape(1, D))

    batch3 = batch.reshape(NBLKS, 1, RBLK)
    return _tc_head(batch3, h0, h1, W_h1, b_h1.reshape(1, 512),
                    W_h2, b_h2.reshape(1, 1))
